# baseline (speedup 1.0000x reference)
import jax
import jax.numpy as jnp
from jax import lax
from jax.experimental import pallas as pl
from jax.experimental.pallas import tpu as pltpu
from jax.experimental.pallas import tpu_sc as plsc

_NC = 2
_NS = 16
_NW = _NC * _NS

_BATCH = 16384
_EMBED = 64
_ROW = 2 * _EMBED
_B_PER_W = _BATCH // _NW
_CHUNK = 256
_N_CHUNKS = _B_PER_W // _CHUNK
_LANES = 16

_VOCAB = 100001
_BLK = 128
_LAST_BLK = (_VOCAB // _BLK) - 1
_N_ITERS = 25
_TAIL_ROW0 = (_LAST_BLK + 1) * _BLK
_TAIL_ROWS = 40
_COMB_ROWS = _TAIL_ROW0 + _TAIL_ROWS


def _transpose_block(src_u, src_m, dst, rows):
    def row_fn(i, _):
        cols = jnp.full((_LANES,), i, jnp.int32)
        for g in range(4):
            dst[i, pl.ds(g * _LANES, _LANES)] = plsc.load_gather(
                src_u, [rows[g], cols])
        for g in range(4):
            dst[i, pl.ds(_EMBED + g * _LANES, _LANES)] = plsc.load_gather(
                src_m, [rows[g], cols])
        return 0

    lax.fori_loop(0, _BLK, row_fn, 0)


def _sc_combine_body(uT_hbm, mT_hbm, tail_hbm, comb_hbm,
                     ublk0, ublk1, mblk0, mblk1, xblk0, xblk1,
                     sem_in0, sem_in1, sem_out0, sem_out1):
    wid = lax.axis_index("s") * _NC + lax.axis_index("c")
    ublk = (ublk0, ublk1)
    mblk = (mblk0, mblk1)
    xblk = (xblk0, xblk1)
    sem_in = (sem_in0, sem_in1)
    sem_out = (sem_out0, sem_out1)
    iota = lax.iota(jnp.int32, _LANES)
    rows = [iota + g * _LANES for g in range(4)]

    def r0_of(kk):
        blk = jnp.minimum(wid + _NW * kk, _LAST_BLK)
        return pl.multiple_of(blk * _BLK, _BLK)

    def start_in(kk):
        p = kk % 2
        r0 = r0_of(kk)
        return (pltpu.async_copy(uT_hbm.at[:, pl.ds(r0, _BLK)],
                                 ublk[p].at[:, pl.ds(0, _BLK)], sem_in[p]),
                pltpu.async_copy(mT_hbm.at[:, pl.ds(r0, _BLK)],
                                 mblk[p].at[:, pl.ds(0, _BLK)], sem_in[p]))

    in_descs = [start_in(0)]
    out_descs = []
    for kk in range(_N_ITERS):
        p = kk % 2
        if kk + 1 < _N_ITERS:
            in_descs.append(start_in(kk + 1))
        du, dm = in_descs[kk]
        du.wait()
        dm.wait()
        if kk >= 2:
            out_descs[kk - 2].wait()
        _transpose_block(ublk[p], mblk[p], xblk[p], rows)
        out_descs.append(
            pltpu.async_copy(xblk[p], comb_hbm.at[pl.ds(r0_of(kk), _BLK)],
                             sem_out[p]))
    out_descs[_N_ITERS - 2].wait()
    out_descs[_N_ITERS - 1].wait()

    @pl.when(wid == _NW - 1)
    def _copy_tail():
        pltpu.sync_copy(tail_hbm, xblk0.at[pl.ds(0, _TAIL_ROWS), :])
        pltpu.sync_copy(xblk0.at[pl.ds(0, _TAIL_ROWS), :],
                        comb_hbm.at[pl.ds(_TAIL_ROW0, _TAIL_ROWS)])


def _sc_combine(uT, mT, tail):
    mesh = plsc.VectorSubcoreMesh(core_axis_name="c", subcore_axis_name="s",
                                  num_cores=_NC, num_subcores=_NS)
    return pl.kernel(
        _sc_combine_body,
        out_type=jax.ShapeDtypeStruct((_COMB_ROWS, _ROW), jnp.float32),
        mesh=mesh,
        scratch_types=[
            pltpu.VMEM((_EMBED, _BLK + 1), jnp.float32),
            pltpu.VMEM((_EMBED, _BLK + 1), jnp.float32),
            pltpu.VMEM((_EMBED, _BLK + 1), jnp.float32),
            pltpu.VMEM((_EMBED, _BLK + 1), jnp.float32),
            pltpu.VMEM((_BLK, _ROW), jnp.float32),
            pltpu.VMEM((_BLK, _ROW), jnp.float32),
            pltpu.SemaphoreType.DMA,
            pltpu.SemaphoreType.DMA,
            pltpu.SemaphoreType.DMA,
            pltpu.SemaphoreType.DMA,
        ],
        compiler_params=pltpu.CompilerParams(needs_layout_passes=False),
    )(uT, mT, tail)


def _sc_gather_body(user_idx_hbm, movie_idx_hbm, table_hbm, x_hbm,
                    uidx_v, midx_v, xbuf_v, mbuf_v, sem_u, sem_m):
    wid = lax.axis_index("s") * _NC + lax.axis_index("c")
    base = wid * _B_PER_W
    pltpu.sync_copy(user_idx_hbm.at[pl.ds(base, _B_PER_W)], uidx_v)
    pltpu.sync_copy(movie_idx_hbm.at[pl.ds(base, _B_PER_W)], midx_v)
    for ch in range(_N_CHUNKS):
        off = ch * _CHUNK
        cp_u = pltpu.async_copy(table_hbm.at[uidx_v.at[pl.ds(off, _CHUNK)]],
                                xbuf_v, sem_u)
        cp_m = pltpu.async_copy(table_hbm.at[midx_v.at[pl.ds(off, _CHUNK)]],
                                mbuf_v, sem_m)
        cp_u.wait()
        cp_m.wait()

        def move_row(r, _):
            for c in range(_EMBED // _LANES):
                col = _EMBED + c * _LANES
                xbuf_v[r, pl.ds(col, _LANES)] = mbuf_v[r, pl.ds(col, _LANES)]
            return 0

        lax.fori_loop(0, _CHUNK, move_row, 0)
        pltpu.sync_copy(xbuf_v, x_hbm.at[pl.ds(base + off, _CHUNK)])


def _sc_gather(user_idx, movie_idx, table):
    mesh = plsc.VectorSubcoreMesh(core_axis_name="c", subcore_axis_name="s",
                                  num_cores=_NC, num_subcores=_NS)
    return pl.kernel(
        _sc_gather_body,
        out_type=jax.ShapeDtypeStruct((_BATCH, _ROW), jnp.float32),
        mesh=mesh,
        scratch_types=[
            pltpu.VMEM((_B_PER_W,), jnp.int32),
            pltpu.VMEM((_B_PER_W,), jnp.int32),
            pltpu.VMEM((_CHUNK, _ROW), jnp.float32),
            pltpu.VMEM((_CHUNK, _ROW), jnp.float32),
            pltpu.SemaphoreType.DMA,
            pltpu.SemaphoreType.DMA,
        ],
    )(user_idx, movie_idx, table)


def _mlp_body(x_ref, w1_ref, b1_ref, w2_ref, b2_ref, out_ref):
    x = x_ref[...]
    w1 = w1_ref[...]
    dn = (((1,), (1,)), ((), ()))
    h = lax.dot_general(x, w1, dn, preferred_element_type=jnp.float32)
    h = jnp.maximum(h + b1_ref[...], 0.0)
    y = jnp.sum(h * w2_ref[...], axis=1) + b2_ref[0, 0]
    out_ref[...] = y


def _mlp(x, W1, b1, W2, b2):
    BR = 2048
    grid = (_BATCH // BR,)
    return pl.pallas_call(
        _mlp_body,
        grid=grid,
        in_specs=[
            pl.BlockSpec((BR, _ROW), lambda i: (i, 0)),
            pl.BlockSpec((128, _ROW), lambda i: (0, 0)),
            pl.BlockSpec((1, 128), lambda i: (0, 0)),
            pl.BlockSpec((1, 128), lambda i: (0, 0)),
            pl.BlockSpec((1, 1), lambda i: (0, 0)),
        ],
        out_specs=pl.BlockSpec((BR,), lambda i: (i,)),
        out_shape=jax.ShapeDtypeStruct((_BATCH,), jnp.float32),
    )(x, W1, b1.reshape(1, 128), W2, b2.reshape(1, 1))


@jax.jit
def kernel(user, movie, user_table, movie_table, W1, b1, W2, b2):
    tail = jnp.pad(
        jnp.concatenate([user_table[_TAIL_ROW0:], movie_table[_TAIL_ROW0:]],
                        axis=1),
        ((0, _TAIL_ROWS - (_VOCAB - _TAIL_ROW0)), (0, 0)))
    table = _sc_combine(user_table.T, movie_table.T, tail)
    x = _sc_gather(user, movie, table)
    return _mlp(x, W1, b1, W2, b2)


# speedup vs baseline: 1.0081x; 1.0081x over previous
import jax
import jax.numpy as jnp
from jax import lax
from jax.experimental import pallas as pl
from jax.experimental.pallas import tpu as pltpu
from jax.experimental.pallas import tpu_sc as plsc

_NC = 2
_NS = 16
_NW = _NC * _NS

_BATCH = 16384
_EMBED = 64
_ROW = 2 * _EMBED
_B_PER_W = _BATCH // _NW
_CHUNK = 256
_N_CHUNKS = _B_PER_W // _CHUNK
_LANES = 16

_VOCAB = 100001
_BLK = 128
_LAST_BLK = (_VOCAB // _BLK) - 1
_N_ITERS = 25
_TAIL_ROW0 = (_LAST_BLK + 1) * _BLK
_TAIL_ROWS = 40
_COMB_ROWS = _TAIL_ROW0 + _TAIL_ROWS


def _transpose_block(src_u, src_m, dst, rows):
    def row_fn(i, _):
        cols = jnp.full((_LANES,), i, jnp.int32)
        for g in range(4):
            dst[i, pl.ds(g * _LANES, _LANES)] = plsc.load_gather(
                src_u, [rows[g], cols])
        for g in range(4):
            dst[i, pl.ds(_EMBED + g * _LANES, _LANES)] = plsc.load_gather(
                src_m, [rows[g], cols])
        return 0

    lax.fori_loop(0, _BLK, row_fn, 0, unroll=8)


def _sc_combine_body(uT_hbm, mT_hbm, tail_hbm, comb_hbm,
                     ublk0, ublk1, mblk0, mblk1, xblk0, xblk1,
                     sem_in0, sem_in1, sem_out0, sem_out1):
    wid = lax.axis_index("s") * _NC + lax.axis_index("c")
    ublk = (ublk0, ublk1)
    mblk = (mblk0, mblk1)
    xblk = (xblk0, xblk1)
    sem_in = (sem_in0, sem_in1)
    sem_out = (sem_out0, sem_out1)
    iota = lax.iota(jnp.int32, _LANES)
    rows = [iota + g * _LANES for g in range(4)]

    def r0_of(kk):
        blk = jnp.minimum(wid + _NW * kk, _LAST_BLK)
        return pl.multiple_of(blk * _BLK, _BLK)

    def in_copy(kk, p):
        r0 = r0_of(kk)
        return (pltpu.make_async_copy(uT_hbm.at[:, pl.ds(r0, _BLK)],
                                      ublk[p].at[:, pl.ds(0, _BLK)],
                                      sem_in[p]),
                pltpu.make_async_copy(mT_hbm.at[:, pl.ds(r0, _BLK)],
                                      mblk[p].at[:, pl.ds(0, _BLK)],
                                      sem_in[p]))

    def out_copy(kk, p):
        return pltpu.make_async_copy(xblk[p],
                                     comb_hbm.at[pl.ds(r0_of(kk), _BLK)],
                                     sem_out[p])

    def step(kk, p):
        @pl.when(kk + 1 < _N_ITERS)
        def _start_next():
            for cp in in_copy(kk + 1, 1 - p):
                cp.start()

        for cp in in_copy(kk, p):
            cp.wait()

        @pl.when(kk >= 2)
        def _wait_prev_out():
            out_copy(kk, p).wait()

        _transpose_block(ublk[p], mblk[p], xblk[p], rows)
        out_copy(kk, p).start()

    for cp in in_copy(0, 0):
        cp.start()

    def loop_body(kk, _):
        @pl.when(kk % 2 == 0)
        def _even():
            step(kk, 0)

        @pl.when(kk % 2 == 1)
        def _odd():
            step(kk, 1)

        return 0

    lax.fori_loop(0, _N_ITERS, loop_body, 0)
    out_copy(_N_ITERS - 2, (_N_ITERS - 2) % 2).wait()
    out_copy(_N_ITERS - 1, (_N_ITERS - 1) % 2).wait()

    @pl.when(wid == _NW - 1)
    def _copy_tail():
        pltpu.sync_copy(tail_hbm, xblk0.at[pl.ds(0, _TAIL_ROWS), :])
        pltpu.sync_copy(xblk0.at[pl.ds(0, _TAIL_ROWS), :],
                        comb_hbm.at[pl.ds(_TAIL_ROW0, _TAIL_ROWS)])


def _sc_combine(uT, mT, tail):
    mesh = plsc.VectorSubcoreMesh(core_axis_name="c", subcore_axis_name="s",
                                  num_cores=_NC, num_subcores=_NS)
    return pl.kernel(
        _sc_combine_body,
        out_type=jax.ShapeDtypeStruct((_COMB_ROWS, _ROW), jnp.float32),
        mesh=mesh,
        scratch_types=[
            pltpu.VMEM((_EMBED, _BLK + 1), jnp.float32),
            pltpu.VMEM((_EMBED, _BLK + 1), jnp.float32),
            pltpu.VMEM((_EMBED, _BLK + 1), jnp.float32),
            pltpu.VMEM((_EMBED, _BLK + 1), jnp.float32),
            pltpu.VMEM((_BLK, _ROW), jnp.float32),
            pltpu.VMEM((_BLK, _ROW), jnp.float32),
            pltpu.SemaphoreType.DMA,
            pltpu.SemaphoreType.DMA,
            pltpu.SemaphoreType.DMA,
            pltpu.SemaphoreType.DMA,
        ],
        compiler_params=pltpu.CompilerParams(needs_layout_passes=False),
    )(uT, mT, tail)


def _sc_gather_body(user_idx_hbm, movie_idx_hbm, table_hbm, x_hbm,
                    uidx_v, midx_v, xbuf_v, mbuf_v, sem_u, sem_m):
    wid = lax.axis_index("s") * _NC + lax.axis_index("c")
    base = wid * _B_PER_W
    pltpu.sync_copy(user_idx_hbm.at[pl.ds(base, _B_PER_W)], uidx_v)
    pltpu.sync_copy(movie_idx_hbm.at[pl.ds(base, _B_PER_W)], midx_v)
    for ch in range(_N_CHUNKS):
        off = ch * _CHUNK
        cp_u = pltpu.async_copy(table_hbm.at[uidx_v.at[pl.ds(off, _CHUNK)]],
                                xbuf_v, sem_u)
        cp_m = pltpu.async_copy(table_hbm.at[midx_v.at[pl.ds(off, _CHUNK)]],
                                mbuf_v, sem_m)
        cp_u.wait()
        cp_m.wait()

        def move_row(r, _):
            for c in range(_EMBED // _LANES):
                col = _EMBED + c * _LANES
                xbuf_v[r, pl.ds(col, _LANES)] = mbuf_v[r, pl.ds(col, _LANES)]
            return 0

        lax.fori_loop(0, _CHUNK, move_row, 0)
        pltpu.sync_copy(xbuf_v, x_hbm.at[pl.ds(base + off, _CHUNK)])


def _sc_gather(user_idx, movie_idx, table):
    mesh = plsc.VectorSubcoreMesh(core_axis_name="c", subcore_axis_name="s",
                                  num_cores=_NC, num_subcores=_NS)
    return pl.kernel(
        _sc_gather_body,
        out_type=jax.ShapeDtypeStruct((_BATCH, _ROW), jnp.float32),
        mesh=mesh,
        scratch_types=[
            pltpu.VMEM((_B_PER_W,), jnp.int32),
            pltpu.VMEM((_B_PER_W,), jnp.int32),
            pltpu.VMEM((_CHUNK, _ROW), jnp.float32),
            pltpu.VMEM((_CHUNK, _ROW), jnp.float32),
            pltpu.SemaphoreType.DMA,
            pltpu.SemaphoreType.DMA,
        ],
    )(user_idx, movie_idx, table)


def _mlp_body(x_ref, w1_ref, b1_ref, w2_ref, b2_ref, out_ref):
    x = x_ref[...]
    w1 = w1_ref[...]
    dn = (((1,), (1,)), ((), ()))
    h = lax.dot_general(x, w1, dn, preferred_element_type=jnp.float32)
    h = jnp.maximum(h + b1_ref[...], 0.0)
    y = jnp.sum(h * w2_ref[...], axis=1) + b2_ref[0, 0]
    out_ref[...] = y


def _mlp(x, W1, b1, W2, b2):
    BR = 2048
    grid = (_BATCH // BR,)
    return pl.pallas_call(
        _mlp_body,
        grid=grid,
        in_specs=[
            pl.BlockSpec((BR, _ROW), lambda i: (i, 0)),
            pl.BlockSpec((128, _ROW), lambda i: (0, 0)),
            pl.BlockSpec((1, 128), lambda i: (0, 0)),
            pl.BlockSpec((1, 128), lambda i: (0, 0)),
            pl.BlockSpec((1, 1), lambda i: (0, 0)),
        ],
        out_specs=pl.BlockSpec((BR,), lambda i: (i,)),
        out_shape=jax.ShapeDtypeStruct((_BATCH,), jnp.float32),
    )(x, W1, b1.reshape(1, 128), W2, b2.reshape(1, 1))


@jax.jit
def kernel(user, movie, user_table, movie_table, W1, b1, W2, b2):
    tail = jnp.pad(
        jnp.concatenate([user_table[_TAIL_ROW0:], movie_table[_TAIL_ROW0:]],
                        axis=1),
        ((0, _TAIL_ROWS - (_VOCAB - _TAIL_ROW0)), (0, 0)))
    table = _sc_combine(user_table.T, movie_table.T, tail)
    x = _sc_gather(user, movie, table)
    return _mlp(x, W1, b1, W2, b2)


# speedup vs baseline: 2.6094x; 2.5885x over previous
import jax
import jax.numpy as jnp
from jax import lax
from jax.experimental import pallas as pl
from jax.experimental.pallas import tpu as pltpu
from jax.experimental.pallas import tpu_sc as plsc

_NC = 2
_NS = 16
_NW = _NC * _NS

_BATCH = 16384
_EMBED = 64
_ROW = 2 * _EMBED
_B_PER_W = _BATCH // _NW
_CHUNK = 128
_N_CHUNKS = _B_PER_W // _CHUNK
_LANES = 16


def _sc_gather_body(user_idx_hbm, movie_idx_hbm, table_hbm, x_hbm,
                    uidx_v, midx_v, xbuf0, xbuf1, mbuf0, mbuf1,
                    sem_in0, sem_in1, sem_out0, sem_out1):
    wid = lax.axis_index("s") * _NC + lax.axis_index("c")
    base = wid * _B_PER_W
    xbuf = (xbuf0, xbuf1)
    mbuf = (mbuf0, mbuf1)
    sem_in = (sem_in0, sem_in1)
    sem_out = (sem_out0, sem_out1)
    pltpu.sync_copy(user_idx_hbm.at[pl.ds(base, _B_PER_W)], uidx_v)
    pltpu.sync_copy(movie_idx_hbm.at[pl.ds(base, _B_PER_W)], midx_v)

    def start_in(ch):
        p = ch % 2
        off = ch * _CHUNK
        return (pltpu.async_copy(table_hbm.at[uidx_v.at[pl.ds(off, _CHUNK)]],
                                 xbuf[p], sem_in[p]),
                pltpu.async_copy(table_hbm.at[midx_v.at[pl.ds(off, _CHUNK)]],
                                 mbuf[p], sem_in[p]))

    in_descs = [start_in(0)]
    out_descs = []
    for ch in range(_N_CHUNKS):
        p = ch % 2
        if ch + 1 < _N_CHUNKS:
            in_descs.append(start_in(ch + 1))
        du, dm = in_descs[ch]
        du.wait()
        dm.wait()
        if ch >= 2:
            out_descs[ch - 2].wait()

        def move_row(r, _):
            for c in range(_EMBED // _LANES):
                col = _EMBED + c * _LANES
                xbuf[p][r, pl.ds(col, _LANES)] = mbuf[p][r, pl.ds(col, _LANES)]
            return 0

        lax.fori_loop(0, _CHUNK, move_row, 0, unroll=4)
        out_descs.append(
            pltpu.async_copy(xbuf[p],
                             x_hbm.at[pl.ds(base + ch * _CHUNK, _CHUNK)],
                             sem_out[p]))
    out_descs[_N_CHUNKS - 2].wait()
    out_descs[_N_CHUNKS - 1].wait()


def _sc_gather(user_idx, movie_idx, table):
    mesh = plsc.VectorSubcoreMesh(core_axis_name="c", subcore_axis_name="s",
                                  num_cores=_NC, num_subcores=_NS)
    return pl.kernel(
        _sc_gather_body,
        out_type=jax.ShapeDtypeStruct((_BATCH, _ROW), jnp.float32),
        mesh=mesh,
        scratch_types=[
            pltpu.VMEM((_B_PER_W,), jnp.int32),
            pltpu.VMEM((_B_PER_W,), jnp.int32),
            pltpu.VMEM((_CHUNK, _ROW), jnp.float32),
            pltpu.VMEM((_CHUNK, _ROW), jnp.float32),
            pltpu.VMEM((_CHUNK, _ROW), jnp.float32),
            pltpu.VMEM((_CHUNK, _ROW), jnp.float32),
            pltpu.SemaphoreType.DMA,
            pltpu.SemaphoreType.DMA,
            pltpu.SemaphoreType.DMA,
            pltpu.SemaphoreType.DMA,
        ],
    )(user_idx, movie_idx, table)


def _mlp_body(x_ref, w1_ref, b1_ref, w2_ref, b2_ref, out_ref):
    x = x_ref[...]
    w1 = w1_ref[...]
    dn = (((1,), (1,)), ((), ()))
    h = lax.dot_general(x, w1, dn, preferred_element_type=jnp.float32)
    h = jnp.maximum(h + b1_ref[...], 0.0)
    y = jnp.sum(h * w2_ref[...], axis=1) + b2_ref[0, 0]
    out_ref[...] = y


def _mlp(x, W1, b1, W2, b2):
    BR = 2048
    grid = (_BATCH // BR,)
    return pl.pallas_call(
        _mlp_body,
        grid=grid,
        in_specs=[
            pl.BlockSpec((BR, _ROW), lambda i: (i, 0)),
            pl.BlockSpec((128, _ROW), lambda i: (0, 0)),
            pl.BlockSpec((1, 128), lambda i: (0, 0)),
            pl.BlockSpec((1, 128), lambda i: (0, 0)),
            pl.BlockSpec((1, 1), lambda i: (0, 0)),
        ],
        out_specs=pl.BlockSpec((BR,), lambda i: (i,)),
        out_shape=jax.ShapeDtypeStruct((_BATCH,), jnp.float32),
    )(x, W1, b1.reshape(1, 128), W2, b2.reshape(1, 1))


@jax.jit
def kernel(user, movie, user_table, movie_table, W1, b1, W2, b2):
    table = jnp.concatenate([user_table, movie_table], axis=1)
    x = _sc_gather(user, movie, table)
    return _mlp(x, W1, b1, W2, b2)


# speedup vs baseline: 2.6594x; 1.0192x over previous
import jax
import jax.numpy as jnp
from jax import lax
from jax.experimental import pallas as pl
from jax.experimental.pallas import tpu as pltpu
from jax.experimental.pallas import tpu_sc as plsc

_NC = 2
_NS = 16
_NW = _NC * _NS

_BATCH = 16384
_EMBED = 64
_ROW = 2 * _EMBED
_B_PER_W = _BATCH // _NW
_CHUNK = 256
_N_CHUNKS = _B_PER_W // _CHUNK
_LANES = 16


def _sc_gather_body(user_idx_hbm, movie_idx_hbm, table_hbm, x_hbm,
                    uidx_v, midx_v, xbuf_v, mbuf_v, sem_u, sem_m):
    wid = lax.axis_index("s") * _NC + lax.axis_index("c")
    base = wid * _B_PER_W
    pltpu.sync_copy(user_idx_hbm.at[pl.ds(base, _B_PER_W)], uidx_v)
    pltpu.sync_copy(movie_idx_hbm.at[pl.ds(base, _B_PER_W)], midx_v)
    for ch in range(_N_CHUNKS):
        off = ch * _CHUNK
        cp_u = pltpu.async_copy(table_hbm.at[uidx_v.at[pl.ds(off, _CHUNK)]],
                                xbuf_v, sem_u)
        cp_m = pltpu.async_copy(table_hbm.at[midx_v.at[pl.ds(off, _CHUNK)]],
                                mbuf_v, sem_m)
        cp_u.wait()
        cp_m.wait()

        def move_row(r, _):
            for c in range(_EMBED // _LANES):
                col = _EMBED + c * _LANES
                xbuf_v[r, pl.ds(col, _LANES)] = mbuf_v[r, pl.ds(col, _LANES)]
            return 0

        lax.fori_loop(0, _CHUNK, move_row, 0)
        pltpu.sync_copy(xbuf_v, x_hbm.at[pl.ds(base + off, _CHUNK)])


def _sc_gather(user_idx, movie_idx, table):
    mesh = plsc.VectorSubcoreMesh(core_axis_name="c", subcore_axis_name="s",
                                  num_cores=_NC, num_subcores=_NS)
    return pl.kernel(
        _sc_gather_body,
        out_type=jax.ShapeDtypeStruct((_BATCH, _ROW), jnp.float32),
        mesh=mesh,
        scratch_types=[
            pltpu.VMEM((_B_PER_W,), jnp.int32),
            pltpu.VMEM((_B_PER_W,), jnp.int32),
            pltpu.VMEM((_CHUNK, _ROW), jnp.float32),
            pltpu.VMEM((_CHUNK, _ROW), jnp.float32),
            pltpu.SemaphoreType.DMA,
            pltpu.SemaphoreType.DMA,
        ],
    )(user_idx, movie_idx, table)


def _mlp_body(x_ref, w1_ref, b1_ref, w2_ref, b2_ref, out_ref):
    x = x_ref[...]
    w1 = w1_ref[...]
    dn = (((1,), (1,)), ((), ()))
    h = lax.dot_general(x, w1, dn, preferred_element_type=jnp.float32)
    h = jnp.maximum(h + b1_ref[...], 0.0)
    y = jnp.sum(h * w2_ref[...], axis=1) + b2_ref[0, 0]
    out_ref[...] = y


def _mlp(x, W1, b1, W2, b2):
    BR = 4096
    grid = (_BATCH // BR,)
    return pl.pallas_call(
        _mlp_body,
        grid=grid,
        in_specs=[
            pl.BlockSpec((BR, _ROW), lambda i: (i, 0)),
            pl.BlockSpec((128, _ROW), lambda i: (0, 0)),
            pl.BlockSpec((1, 128), lambda i: (0, 0)),
            pl.BlockSpec((1, 128), lambda i: (0, 0)),
            pl.BlockSpec((1, 1), lambda i: (0, 0)),
        ],
        out_specs=pl.BlockSpec((BR,), lambda i: (i,)),
        out_shape=jax.ShapeDtypeStruct((_BATCH,), jnp.float32),
    )(x, W1, b1.reshape(1, 128), W2, b2.reshape(1, 1))


@jax.jit
def kernel(user, movie, user_table, movie_table, W1, b1, W2, b2):
    table = jnp.concatenate([user_table, movie_table], axis=1)
    x = _sc_gather(user, movie, table)
    return _mlp(x, W1, b1, W2, b2)
